# baseline (speedup 1.0000x reference)
import functools

import jax
import jax.numpy as jnp
from jax import lax
from jax.experimental import pallas as pl
from jax.experimental.pallas import tpu as pltpu
from jax.experimental.pallas import tpu_sc as plsc

_NC = 2
_NS = 16
_L = 16
_CH = 128
_BLK = 512


def _deg_kernel(n_pad, gpt):
    nw = _NC * _NS
    mesh = plsc.VectorSubcoreMesh(core_axis_name="c", subcore_axis_name="s")

    @functools.partial(
        pl.kernel,
        out_type=jax.ShapeDtypeStruct((nw, n_pad), jnp.float32),
        mesh=mesh,
        scratch_types=[
            pltpu.VMEM((gpt, _L), jnp.int32),
            pltpu.VMEM((n_pad,), jnp.float32),
        ],
        compiler_params=pltpu.CompilerParams(needs_layout_passes=False),
    )
    def body(dst_hbm, part_hbm, idx_v, hist_v):
        c = lax.axis_index("c")
        s = lax.axis_index("s")
        wid = s * _NC + c
        pltpu.sync_copy(dst_hbm.at[wid], idx_v)
        zeros = jnp.zeros((_L,), jnp.float32)

        def zero_body(i, carry):
            hist_v[pl.ds(i * _L, _L)] = zeros
            return carry

        lax.fori_loop(0, n_pad // _L, zero_body, 0)
        ones = jnp.ones((_L,), jnp.float32)

        def acc_body(i, carry):
            idx = idx_v[i, :]
            plsc.addupdate_scatter(hist_v, [idx], ones)
            return carry

        lax.fori_loop(0, gpt, acc_body, 0)
        pltpu.sync_copy(hist_v, part_hbm.at[wid])

    return body


def _scatter_kernel(n_pad, cpt, half):
    rpt = n_pad // _NS
    zb = rpt // _CH
    mesh = plsc.VectorSubcoreMesh(core_axis_name="c", subcore_axis_name="s")

    @functools.partial(
        pl.kernel,
        out_type=jax.ShapeDtypeStruct((_NC, n_pad, half), jnp.float32),
        mesh=mesh,
        scratch_types=[
            pltpu.VMEM((cpt, _CH), jnp.int32),
            pltpu.VMEM((8, _CH), jnp.int32),
            pltpu.VMEM((_CH, half), jnp.float32),
            pltpu.VMEM((_CH, half), jnp.float32),
            pltpu.VMEM_SHARED((n_pad, half), jnp.float32),
            pltpu.SemaphoreType.DMA,
            pltpu.SemaphoreType.DMA,
            pltpu.SemaphoreType.DMA,
            pltpu.SemaphoreType.DMA,
        ],
        compiler_params=pltpu.CompilerParams(needs_layout_passes=False),
    )
    def body(g_hbm, src_hbm, dst_hbm, out_hbm, didx, sidx_blk,
             buf_a, buf_b, acc, sem_a, sem_b, sem_sa, sem_sb):
        c = lax.axis_index("c")
        s = lax.axis_index("s")
        zeros = jnp.zeros((_L,), jnp.float32)

        def zrow(i, carry):
            for k in range(half // _L):
                buf_a[i, pl.ds(k * _L, _L)] = zeros
            return carry

        lax.fori_loop(0, _CH, zrow, 0)
        for t in range(zb):
            pltpu.sync_copy(buf_a, acc.at[pl.ds(s * rpt + t * _CH, _CH)])
        pltpu.sync_copy(dst_hbm.at[s], didx)
        plsc.subcore_barrier()

        def blk(bi, carry):
            pltpu.sync_copy(src_hbm.at[c, s, pl.ds(bi * 8, 8)], sidx_blk)

            def group(hi, carry2):
                j0 = bi * 8 + hi * 2
                da = pltpu.async_copy(
                    g_hbm.at[sidx_blk.at[hi * 2]], buf_a, sem_a)
                db = pltpu.async_copy(
                    g_hbm.at[sidx_blk.at[hi * 2 + 1]], buf_b, sem_b)
                da.wait()
                pltpu.sync_copy(buf_a, acc.at[didx.at[j0]], add=True)
                db.wait()
                pltpu.sync_copy(buf_b, acc.at[didx.at[j0 + 1]], add=True)
                return carry2

            lax.fori_loop(0, 4, group, 0)
            return carry

        lax.fori_loop(0, cpt // 8, blk, 0)
        plsc.subcore_barrier()
        pltpu.sync_copy(acc.at[pl.ds(s * rpt, rpt)],
                        out_hbm.at[c, pl.ds(s * rpt, rpt)])

    return body


def _g_tc(x_pad, w, partials, n_pad, d):
    half = d // 2
    nb = n_pad // _BLK
    nw = _NC * _NS

    def body(x_ref, w_ref, p_ref, o_ref):
        deg = jnp.sum(p_ref[...], axis=0) + 1.0
        dinv = lax.rsqrt(deg)
        h = jnp.dot(x_ref[...], w_ref[...], preferred_element_type=jnp.float32)
        o_ref[0] = h * dinv[:, None]

    return pl.pallas_call(
        body,
        grid=(nb, _NC),
        in_specs=[
            pl.BlockSpec((_BLK, d), lambda i, c: (i, 0)),
            pl.BlockSpec((d, half), lambda i, c: (0, c)),
            pl.BlockSpec((nw, _BLK), lambda i, c: (0, i)),
        ],
        out_specs=pl.BlockSpec((1, _BLK, half), lambda i, c: (c, i, 0)),
        out_shape=jax.ShapeDtypeStruct((_NC, n_pad, half), jnp.float32),
    )(x_pad, w, partials)


def _combine_tc(sacc, g, partials, b2, n, d, n_pad):
    half = d // 2
    nb = n_pad // _BLK
    nw = _NC * _NS

    def body(s_ref, g_ref, p_ref, b_ref, o_ref):
        deg = jnp.sum(p_ref[...], axis=0) + 1.0
        dinv = lax.rsqrt(deg)
        o_ref[...] = (s_ref[0] + g_ref[0]) * dinv[:, None] + b_ref[0, 0]

    return pl.pallas_call(
        body,
        grid=(nb, _NC),
        in_specs=[
            pl.BlockSpec((1, _BLK, half), lambda i, c: (c, i, 0)),
            pl.BlockSpec((1, _BLK, half), lambda i, c: (c, i, 0)),
            pl.BlockSpec((nw, _BLK), lambda i, c: (0, i)),
            pl.BlockSpec((1, 1, half), lambda i, c: (c, 0, 0)),
        ],
        out_specs=pl.BlockSpec((_BLK, half), lambda i, c: (i, c)),
        out_shape=jax.ShapeDtypeStruct((n, d), jnp.float32),
    )(sacc, g, partials, b2)


def kernel(x, edge_index, W, b):
    n, d = x.shape
    e = edge_index.shape[1]
    half = d // 2
    nw = _NC * _NS
    n_align = _NS * _CH
    e_align = _NS * _CH * 8
    n_pad = -(-(n + 1) // n_align) * n_align
    e_pad = -(-e // e_align) * e_align
    cpt = e_pad // (_NS * _CH)
    gpt = e_pad // (nw * _L)

    src = edge_index[0]
    dst = edge_index[1]
    pad_ids = jnp.arange(e_pad - e, dtype=jnp.int32)
    src_p = jnp.concatenate([src, n + pad_ids % (n_pad - n)])
    dst_p = jnp.concatenate([dst, n + 1 + pad_ids % (n_pad - n - 1)])

    partials = _deg_kernel(n_pad, gpt)(dst_p.reshape(nw, gpt, _L))
    x_pad = jnp.pad(x, ((0, n_pad - n), (0, 0)))
    g = _g_tc(x_pad, W, partials, n_pad, d)

    src2 = jnp.stack([src_p, src_p + n_pad]).reshape(_NC, _NS, cpt, _CH)
    dst3 = dst_p.reshape(_NS, cpt, _CH)
    gflat = g.reshape(_NC * n_pad, half)
    sacc = _scatter_kernel(n_pad, cpt, half)(gflat, src2, dst3)

    return _combine_tc(sacc, g, partials, b.reshape(_NC, 1, half), n, d, n_pad)


# speedup vs baseline: 1.0113x; 1.0113x over previous
import functools

import jax
import jax.numpy as jnp
from jax import lax
from jax.experimental import pallas as pl
from jax.experimental.pallas import tpu as pltpu
from jax.experimental.pallas import tpu_sc as plsc

_NC = 2
_NS = 16
_L = 16
_CH = 128
_BLK = 512


def _deg_kernel(n_pad, gpt):
    nw = _NC * _NS
    mesh = plsc.VectorSubcoreMesh(core_axis_name="c", subcore_axis_name="s")

    @functools.partial(
        pl.kernel,
        out_type=jax.ShapeDtypeStruct((nw, n_pad), jnp.float32),
        mesh=mesh,
        scratch_types=[
            pltpu.VMEM((gpt, _L), jnp.int32),
            pltpu.VMEM((n_pad,), jnp.float32),
        ],
        compiler_params=pltpu.CompilerParams(needs_layout_passes=False),
    )
    def body(dst_hbm, part_hbm, idx_v, hist_v):
        c = lax.axis_index("c")
        s = lax.axis_index("s")
        wid = s * _NC + c
        pltpu.sync_copy(dst_hbm.at[wid], idx_v)
        zeros = jnp.zeros((_L,), jnp.float32)

        def zero_body(i, carry):
            hist_v[pl.ds(i * _L, _L)] = zeros
            return carry

        lax.fori_loop(0, n_pad // _L, zero_body, 0)
        ones = jnp.ones((_L,), jnp.float32)

        def acc_body(i, carry):
            idx = idx_v[i, :]
            plsc.addupdate_scatter(hist_v, [idx], ones)
            return carry

        lax.fori_loop(0, gpt, acc_body, 0)
        pltpu.sync_copy(hist_v, part_hbm.at[wid])

    return body


def _scatter_kernel(n_pad, cpt, half):
    rpt = n_pad // _NS
    zb = rpt // _CH
    mesh = plsc.VectorSubcoreMesh(core_axis_name="c", subcore_axis_name="s")

    @functools.partial(
        pl.kernel,
        out_type=jax.ShapeDtypeStruct((_NC, n_pad, half), jnp.float32),
        mesh=mesh,
        scratch_types=[
            pltpu.VMEM((cpt, _CH), jnp.int32),
            pltpu.VMEM((8, _CH), jnp.int32),
            pltpu.VMEM((_CH, half), jnp.float32),
            pltpu.VMEM((_CH, half), jnp.float32),
            pltpu.VMEM_SHARED((n_pad, half), jnp.float32),
            pltpu.SemaphoreType.DMA,
            pltpu.SemaphoreType.DMA,
            pltpu.SemaphoreType.DMA,
            pltpu.SemaphoreType.DMA,
        ],
        compiler_params=pltpu.CompilerParams(needs_layout_passes=False),
    )
    def body(g_hbm, src_hbm, dst_hbm, out_hbm, didx, sidx_blk,
             buf_a, buf_b, acc, sem_a, sem_b, sem_sa, sem_sb):
        c = lax.axis_index("c")
        s = lax.axis_index("s")
        zeros = jnp.zeros((_L,), jnp.float32)

        def zrow(i, carry):
            for k in range(half // _L):
                buf_a[i, pl.ds(k * _L, _L)] = zeros
            return carry

        lax.fori_loop(0, _CH, zrow, 0)
        for t in range(zb):
            pltpu.sync_copy(buf_a, acc.at[pl.ds(s * rpt + t * _CH, _CH)])
        pltpu.sync_copy(dst_hbm.at[s], didx)
        plsc.subcore_barrier()

        def blk(bi, carry):
            pltpu.sync_copy(src_hbm.at[s, pl.ds(bi * 8, 8)], sidx_blk)

            def group(hi, carry2):
                j0 = bi * 8 + hi * 2
                da = pltpu.async_copy(
                    g_hbm.at[c].at[sidx_blk.at[hi * 2]], buf_a, sem_a)
                db = pltpu.async_copy(
                    g_hbm.at[c].at[sidx_blk.at[hi * 2 + 1]], buf_b, sem_b)
                da.wait()
                pltpu.sync_copy(buf_a, acc.at[didx.at[j0]], add=True)
                db.wait()
                pltpu.sync_copy(buf_b, acc.at[didx.at[j0 + 1]], add=True)
                return carry2

            lax.fori_loop(0, 4, group, 0)
            return carry

        lax.fori_loop(0, cpt // 8, blk, 0)
        plsc.subcore_barrier()
        pltpu.sync_copy(acc.at[pl.ds(s * rpt, rpt)],
                        out_hbm.at[c, pl.ds(s * rpt, rpt)])

    return body


def _g_tc(x_pad, w, partials, n_pad, d):
    half = d // 2
    nb = n_pad // _BLK
    nw = _NC * _NS

    def body(x_ref, w_ref, p_ref, o_ref):
        deg = jnp.sum(p_ref[...], axis=0) + 1.0
        dinv = lax.rsqrt(deg)
        h = jnp.dot(x_ref[...], w_ref[...], preferred_element_type=jnp.float32)
        o_ref[0] = h * dinv[:, None]

    return pl.pallas_call(
        body,
        grid=(nb, _NC),
        in_specs=[
            pl.BlockSpec((_BLK, d), lambda i, c: (i, 0)),
            pl.BlockSpec((d, half), lambda i, c: (0, c)),
            pl.BlockSpec((nw, _BLK), lambda i, c: (0, i)),
        ],
        out_specs=pl.BlockSpec((1, _BLK, half), lambda i, c: (c, i, 0)),
        out_shape=jax.ShapeDtypeStruct((_NC, n_pad, half), jnp.float32),
    )(x_pad, w, partials)


def _combine_tc(sacc, g, partials, b2, n, d, n_pad):
    half = d // 2
    nb = n_pad // _BLK
    nw = _NC * _NS

    def body(s_ref, g_ref, p_ref, b_ref, o_ref):
        deg = jnp.sum(p_ref[...], axis=0) + 1.0
        dinv = lax.rsqrt(deg)
        o_ref[...] = (s_ref[0] + g_ref[0]) * dinv[:, None] + b_ref[0, 0]

    return pl.pallas_call(
        body,
        grid=(nb, _NC),
        in_specs=[
            pl.BlockSpec((1, _BLK, half), lambda i, c: (c, i, 0)),
            pl.BlockSpec((1, _BLK, half), lambda i, c: (c, i, 0)),
            pl.BlockSpec((nw, _BLK), lambda i, c: (0, i)),
            pl.BlockSpec((1, 1, half), lambda i, c: (c, 0, 0)),
        ],
        out_specs=pl.BlockSpec((_BLK, half), lambda i, c: (i, c)),
        out_shape=jax.ShapeDtypeStruct((n, d), jnp.float32),
    )(sacc, g, partials, b2)


def kernel(x, edge_index, W, b):
    n, d = x.shape
    e = edge_index.shape[1]
    half = d // 2
    nw = _NC * _NS
    n_align = _NS * _CH
    e_align = _NS * _CH * 8
    n_pad = -(-(n + 1) // n_align) * n_align
    e_pad = -(-e // e_align) * e_align
    cpt = e_pad // (_NS * _CH)
    gpt = e_pad // (nw * _L)

    src = edge_index[0]
    dst = edge_index[1]
    pad_ids = jnp.arange(e_pad - e, dtype=jnp.int32)
    src_p = jnp.concatenate([src, n + pad_ids % (n_pad - n)])
    dst_p = jnp.concatenate([dst, n + 1 + pad_ids % (n_pad - n - 1)])

    partials = _deg_kernel(n_pad, gpt)(dst_p.reshape(nw, gpt, _L))
    g = _g_tc(x, W, partials, n_pad, d)

    src3 = src_p.reshape(_NS, cpt, _CH)
    dst3 = dst_p.reshape(_NS, cpt, _CH)
    sacc = _scatter_kernel(n_pad, cpt, half)(g, src3, dst3)

    return _combine_tc(sacc, g, partials, b.reshape(_NC, 1, half), n, d, n_pad)
